# baseline (speedup 1.0000x reference)
import functools
import jax
import jax.numpy as jnp
from jax import lax
from jax.experimental import pallas as pl
from jax.experimental.pallas import tpu as pltpu
from jax.experimental.pallas import tpu_sc as plsc

NC = 2
NS = 16
NW = NC * NS

ROUND = 256
PACK = 4
LANES = 128
D = 32
GROUPS = ROUND // 16
FIELDS = 100
FPAD = 104


def _make_kernel(total, vocab):
    per_w = total // NW
    idx_rows_per_round = 2
    n_round = per_w // (idx_rows_per_round * FIELDS)
    idx_rows_per_w = per_w // FIELDS
    batch = total // FIELDS
    out_rows = batch * FPAD * PACK

    mesh = plsc.VectorSubcoreMesh(core_axis_name="c", subcore_axis_name="s")

    @functools.partial(
        pl.kernel,
        out_type=jax.ShapeDtypeStruct((out_rows, D), jnp.float32),
        mesh=mesh,
        scratch_types=[
            pltpu.VMEM((idx_rows_per_round, LANES), jnp.int32),
            pltpu.VMEM((idx_rows_per_round, LANES), jnp.int32),
            pltpu.VMEM((idx_rows_per_round, LANES), jnp.int32),
            pltpu.VMEM((idx_rows_per_round, LANES), jnp.int32),
            pltpu.VMEM((idx_rows_per_round, LANES), jnp.int32),
            pltpu.VMEM((idx_rows_per_round, LANES), jnp.int32),
            pltpu.VMEM((idx_rows_per_round, LANES), jnp.int32),
            pltpu.VMEM((idx_rows_per_round, LANES), jnp.int32),
            pltpu.VMEM((ROUND, LANES), jnp.float32),
            pltpu.VMEM((ROUND, LANES), jnp.float32),
            pltpu.VMEM((ROUND, D), jnp.float32),
            pltpu.VMEM((ROUND, D), jnp.float32),
            pltpu.SemaphoreType.DMA,
            pltpu.SemaphoreType.DMA,
            pltpu.SemaphoreType.DMA,
            pltpu.SemaphoreType.DMA,
            pltpu.SemaphoreType.DMA,
            pltpu.SemaphoreType.DMA,
        ],
        compiler_params=pltpu.CompilerParams(
            use_tc_tiling_on_sc=False, needs_layout_passes=False),
    )
    def body(idx_hbm, table_hbm, out_hbm,
             idxb0, idxb1, pidx0, pidx1, ofb0, ofb1, sidx0, sidx1,
             pk0, pk1, ob0, ob1, si0, si1, sg0, sg1, so0, so1):
        wid = lax.axis_index("s") * NC + lax.axis_index("c")
        ibase = wid * idx_rows_per_w
        osbase = wid * (per_w // FIELDS) * FPAD * PACK
        idxb = (idxb0, idxb1)
        pidx = (pidx0, pidx1)
        ofb = (ofb0, ofb1)
        sidx = (sidx0, sidx1)
        pk = (pk0, pk1)
        ob = (ob0, ob1)
        si = (si0, si1)
        sg = (sg0, sg1)
        so = (so0, so1)
        lane = lax.iota(jnp.int32, 16)

        def fire_idx(r, slot):
            pltpu.async_copy(
                idx_hbm.at[pl.ds(ibase + r * idx_rows_per_round,
                                 idx_rows_per_round)],
                idxb[slot], si[slot])

        def wait_idx(slot):
            pltpu.make_async_copy(
                idx_hbm.at[pl.ds(0, idx_rows_per_round)], idxb[slot],
                si[slot]).wait()

        dump_row = osbase + FIELDS * PACK

        def prep(r, slot):
            for q in range(idx_rows_per_round):
                for g in range(LANES // 16):
                    v = idxb[slot][q, pl.ds(g * 16, 16)]
                    pidx[slot][q, pl.ds(g * 16, 16)] = v >> 2
                    ofb[slot][q, pl.ds(g * 16, 16)] = (v & 3) * D
                    nvalid = FIELDS - g * 16
                    if nvalid >= 16:
                        x = lane + ((r * idx_rows_per_round + q) * FIELDS
                                    + g * 16)
                        bl = ((x >> 2) * 20972) >> 19
                        srow = (osbase + x * PACK
                                + bl * ((FPAD - FIELDS) * PACK))
                    elif nvalid > 0:
                        x = lane + ((r * idx_rows_per_round + q) * FIELDS
                                    + g * 16)
                        bl = ((x >> 2) * 20972) >> 19
                        srow = jnp.where(
                            lane < nvalid,
                            osbase + x * PACK
                            + bl * ((FPAD - FIELDS) * PACK),
                            dump_row)
                    else:
                        srow = jnp.full((16,), dump_row, jnp.int32)
                    sidx[slot][q, pl.ds(g * 16, 16)] = srow

        def fire_gathers(slot):
            for s in range(idx_rows_per_round):
                pltpu.async_copy(
                    table_hbm.at[pidx[slot].at[s]],
                    pk[slot].at[pl.ds(s * LANES, LANES)], sg[slot])

        def wait_gathers(slot):
            pltpu.make_async_copy(
                table_hbm.at[pl.ds(0, ROUND)], pk[slot], sg[slot]).wait()

        def extract(slot):
            def group(g, _):
                off = ofb[slot][g // 8, pl.ds((g % 8) * 16, 16)]
                lj = lane + g * 16
                for c in range(D):
                    rot = (lane + c) & (D - 1)
                    vals = plsc.load_gather(pk[slot], [lj, off + rot])
                    plsc.store_scatter(ob[slot], [lj, rot], vals)
                return 0

            lax.fori_loop(0, GROUPS, group, 0)

        def fire_out(slot):
            for s in range(idx_rows_per_round):
                pltpu.async_copy(
                    ob[slot].at[pl.ds(s * LANES, LANES)],
                    out_hbm.at[sidx[slot].at[s]], so[slot])

        def wait_out(slot):
            pltpu.make_async_copy(
                ob[slot], out_hbm.at[pl.ds(0, ROUND)], so[slot]).wait()

        def step(r, slot, other):
            def advance():
                wait_idx(other)
                pl.when(r >= 1)(lambda: wait_out(other))
                prep(r + 1, other)
            pl.when(r + 1 < n_round)(advance)
            wait_gathers(slot)
            pl.when(r + 1 < n_round)(lambda: fire_gathers(other))
            pl.when(r + 2 < n_round)(lambda: fire_idx(r + 2, slot))
            extract(slot)
            fire_out(slot)

        fire_idx(0, 0)
        wait_idx(0)
        prep(0, 0)
        fire_gathers(0)
        fire_idx(1, 1)

        def pair(i, _):
            r0 = i * 2
            step(r0, 0, 1)
            step(r0 + 1, 1, 0)
            return 0

        lax.fori_loop(0, n_round // 2, pair, 0)
        wait_out(0)
        wait_out(1)

    return body


@jax.jit
def kernel(input, weight):
    B, F = input.shape
    V, _ = weight.shape
    total = B * F
    idx = jnp.pad(input.astype(jnp.int32), ((0, 0), (0, LANES - F)))
    wpk = weight.reshape(V // PACK, LANES)
    out = _make_kernel(total, V)(idx, wpk)
    return out.reshape(B, FPAD, PACK * D)[:, :F, :D]


# speedup vs baseline: 1.0029x; 1.0029x over previous
import functools
import jax
import jax.numpy as jnp
from jax import lax
from jax.experimental import pallas as pl
from jax.experimental.pallas import tpu as pltpu
from jax.experimental.pallas import tpu_sc as plsc

NC = 2
NS = 16
NW = NC * NS

ROUND = 256
PACK = 4
LANES = 128
D = 32
GROUPS = ROUND // 16
FIELDS = 100
FPAD = 104


def _make_kernel(total, vocab):
    per_w = total // NW
    idx_rows_per_round = 2
    n_round = per_w // (idx_rows_per_round * FIELDS)
    idx_rows_per_w = per_w // FIELDS
    batch = total // FIELDS
    out_rows = batch * FPAD * PACK

    mesh = plsc.VectorSubcoreMesh(core_axis_name="c", subcore_axis_name="s")

    @functools.partial(
        pl.kernel,
        out_type=jax.ShapeDtypeStruct((out_rows, D), jnp.float32),
        mesh=mesh,
        scratch_types=[
            pltpu.VMEM((idx_rows_per_round, LANES), jnp.int32),
            pltpu.VMEM((idx_rows_per_round, LANES), jnp.int32),
            pltpu.VMEM((idx_rows_per_round, LANES), jnp.int32),
            pltpu.VMEM((idx_rows_per_round, LANES), jnp.int32),
            pltpu.VMEM((idx_rows_per_round, LANES), jnp.int32),
            pltpu.VMEM((idx_rows_per_round, LANES), jnp.int32),
            pltpu.VMEM((idx_rows_per_round, LANES), jnp.int32),
            pltpu.VMEM((idx_rows_per_round, LANES), jnp.int32),
            pltpu.VMEM((ROUND, LANES), jnp.float32),
            pltpu.VMEM((ROUND, LANES), jnp.float32),
            pltpu.VMEM((ROUND, D), jnp.float32),
            pltpu.VMEM((ROUND, D), jnp.float32),
            pltpu.SemaphoreType.DMA,
            pltpu.SemaphoreType.DMA,
            pltpu.SemaphoreType.DMA,
            pltpu.SemaphoreType.DMA,
            pltpu.SemaphoreType.DMA,
            pltpu.SemaphoreType.DMA,
        ],
        compiler_params=pltpu.CompilerParams(
            use_tc_tiling_on_sc=False, needs_layout_passes=False),
    )
    def body(idx_hbm, table_hbm, out_hbm,
             idxb0, idxb1, pidx0, pidx1, ofb0, ofb1, sidx0, sidx1,
             pk0, pk1, ob0, ob1, si0, si1, sg0, sg1, so0, so1):
        wid = lax.axis_index("s") * NC + lax.axis_index("c")
        ibase = wid * idx_rows_per_w
        osbase = wid * (per_w // FIELDS) * FPAD * PACK
        idxb = (idxb0, idxb1)
        pidx = (pidx0, pidx1)
        ofb = (ofb0, ofb1)
        sidx = (sidx0, sidx1)
        pk = (pk0, pk1)
        ob = (ob0, ob1)
        si = (si0, si1)
        sg = (sg0, sg1)
        so = (so0, so1)
        lane = lax.iota(jnp.int32, 16)

        def fire_idx(r, slot):
            pltpu.async_copy(
                idx_hbm.at[pl.ds(ibase + r * idx_rows_per_round,
                                 idx_rows_per_round)],
                idxb[slot], si[slot])

        def wait_idx(slot):
            pltpu.make_async_copy(
                idx_hbm.at[pl.ds(0, idx_rows_per_round)], idxb[slot],
                si[slot]).wait()

        dump0 = osbase + FIELDS * PACK

        def prep(r, slot):
            for q in range(idx_rows_per_round):
                for g in range(LANES // 16):
                    v = idxb[slot][q, pl.ds(g * 16, 16)]
                    pidx[slot][q, pl.ds(g * 16, 16)] = v >> 2
                    ofb[slot][q, pl.ds(g * 16, 16)] = (v & 3) * D
                    nvalid = FIELDS - g * 16
                    if nvalid >= 16:
                        x = lane + ((r * idx_rows_per_round + q) * FIELDS
                                    + g * 16)
                        bl = ((x >> 2) * 20972) >> 19
                        srow = (osbase + x * PACK
                                + bl * ((FPAD - FIELDS) * PACK))
                    elif nvalid > 0:
                        x = lane + ((r * idx_rows_per_round + q) * FIELDS
                                    + g * 16)
                        bl = ((x >> 2) * 20972) >> 19
                        dmp = dump0 + (q * 2) * (FPAD * PACK) + lane
                        srow = jnp.where(
                            lane < nvalid,
                            osbase + x * PACK
                            + bl * ((FPAD - FIELDS) * PACK),
                            dmp)
                    else:
                        srow = dump0 + (q * 2 + 1) * (FPAD * PACK) + lane
                    sidx[slot][q, pl.ds(g * 16, 16)] = srow

        def fire_gathers(slot):
            for s in range(idx_rows_per_round):
                pltpu.async_copy(
                    table_hbm.at[pidx[slot].at[s]],
                    pk[slot].at[pl.ds(s * LANES, LANES)], sg[slot])

        def wait_gathers(slot):
            pltpu.make_async_copy(
                table_hbm.at[pl.ds(0, ROUND)], pk[slot], sg[slot]).wait()

        def extract(slot):
            def group(g, _):
                off = ofb[slot][g // 8, pl.ds((g % 8) * 16, 16)]
                lj = lane + g * 16
                for c in range(D):
                    rot = (lane + c) & (D - 1)
                    vals = plsc.load_gather(pk[slot], [lj, off + rot])
                    plsc.store_scatter(ob[slot], [lj, rot], vals)
                return 0

            lax.fori_loop(0, GROUPS, group, 0)

        def fire_out(slot):
            for s in range(idx_rows_per_round):
                pltpu.async_copy(
                    ob[slot].at[pl.ds(s * LANES, LANES)],
                    out_hbm.at[sidx[slot].at[s]], so[slot])

        def wait_out(slot):
            pltpu.make_async_copy(
                ob[slot], out_hbm.at[pl.ds(0, ROUND)], so[slot]).wait()

        def step(r, slot, other):
            def advance():
                wait_idx(other)
                pl.when(r >= 1)(lambda: wait_out(other))
                prep(r + 1, other)
            pl.when(r + 1 < n_round)(advance)
            wait_gathers(slot)
            pl.when(r + 1 < n_round)(lambda: fire_gathers(other))
            pl.when(r + 2 < n_round)(lambda: fire_idx(r + 2, slot))
            extract(slot)
            fire_out(slot)

        fire_idx(0, 0)
        wait_idx(0)
        prep(0, 0)
        fire_gathers(0)
        fire_idx(1, 1)

        def pair(i, _):
            r0 = i * 2
            step(r0, 0, 1)
            step(r0 + 1, 1, 0)
            return 0

        lax.fori_loop(0, n_round // 2, pair, 0)
        wait_out(0)
        wait_out(1)

    return body


@jax.jit
def kernel(input, weight):
    B, F = input.shape
    V, _ = weight.shape
    total = B * F
    idx = jnp.pad(input.astype(jnp.int32), ((0, 0), (0, LANES - F)))
    wpk = weight.reshape(V // PACK, LANES)
    out = _make_kernel(total, V)(idx, wpk)
    return out.reshape(B, FPAD, PACK * D)[:, :F, :D]


# speedup vs baseline: 12.9869x; 12.9493x over previous
import functools
import jax
import jax.numpy as jnp
from jax import lax
from jax.experimental import pallas as pl
from jax.experimental.pallas import tpu as pltpu
from jax.experimental.pallas import tpu_sc as plsc

NC = 2
NS = 16
NW = NC * NS

ROUND = 256
PACK = 4
LANES = 128
D = 32
GROUPS = ROUND // 16
FIELDS = 100
FPAD = 104


def _make_kernel(total, vocab):
    per_w = total // NW
    idx_rows_per_round = 2
    n_round = per_w // (idx_rows_per_round * FIELDS)
    idx_rows_per_w = per_w // FIELDS
    batch = total // FIELDS
    out_rows = batch * FPAD * PACK

    mesh = plsc.VectorSubcoreMesh(core_axis_name="c", subcore_axis_name="s")

    @functools.partial(
        pl.kernel,
        out_type=jax.ShapeDtypeStruct((out_rows, D), jnp.float32),
        mesh=mesh,
        scratch_types=[
            pltpu.VMEM((idx_rows_per_round, LANES), jnp.int32),
            pltpu.VMEM((idx_rows_per_round, LANES), jnp.int32),
            pltpu.VMEM((idx_rows_per_round, LANES), jnp.int32),
            pltpu.VMEM((idx_rows_per_round, LANES), jnp.int32),
            pltpu.VMEM((idx_rows_per_round, LANES), jnp.int32),
            pltpu.VMEM((idx_rows_per_round, LANES), jnp.int32),
            pltpu.VMEM((idx_rows_per_round, LANES), jnp.int32),
            pltpu.VMEM((idx_rows_per_round, LANES), jnp.int32),
            pltpu.VMEM((ROUND, LANES), jnp.float32),
            pltpu.VMEM((ROUND, LANES), jnp.float32),
            pltpu.VMEM((ROUND, D), jnp.float32),
            pltpu.VMEM((ROUND, D), jnp.float32),
            pltpu.SemaphoreType.DMA,
            pltpu.SemaphoreType.DMA,
            pltpu.SemaphoreType.DMA,
            pltpu.SemaphoreType.DMA,
            pltpu.SemaphoreType.DMA,
            pltpu.SemaphoreType.DMA,
        ],
        compiler_params=pltpu.CompilerParams(
            use_tc_tiling_on_sc=False, needs_layout_passes=False),
    )
    def body(idx_hbm, table_hbm, out_hbm,
             idxb0, idxb1, pidx0, pidx1, ofb0, ofb1, sidx0, sidx1,
             pk0, pk1, ob0, ob1, si0, si1, sg0, sg1, so0, so1):
        wid = lax.axis_index("s") * NC + lax.axis_index("c")
        ibase = wid * idx_rows_per_w
        osbase = wid * (per_w // FIELDS) * FPAD * PACK
        idxb = (idxb0, idxb1)
        pidx = (pidx0, pidx1)
        ofb = (ofb0, ofb1)
        sidx = (sidx0, sidx1)
        pk = (pk0, pk1)
        ob = (ob0, ob1)
        si = (si0, si1)
        sg = (sg0, sg1)
        so = (so0, so1)
        lane = lax.iota(jnp.int32, 16)

        def fire_idx(r, slot):
            pltpu.async_copy(
                idx_hbm.at[pl.ds(ibase + r * idx_rows_per_round,
                                 idx_rows_per_round)],
                idxb[slot], si[slot])

        def wait_idx(slot):
            pltpu.make_async_copy(
                idx_hbm.at[pl.ds(0, idx_rows_per_round)], idxb[slot],
                si[slot]).wait()

        dump0 = osbase + FIELDS * PACK

        def prep(r, slot):
            for q in range(idx_rows_per_round):
                for g in range(LANES // 16):
                    v = idxb[slot][q, pl.ds(g * 16, 16)]
                    nvalid = FIELDS - g * 16
                    if nvalid >= 16:
                        pidx[slot][q, pl.ds(g * 16, 16)] = v >> 2
                    else:
                        dummy = wid * 64 + (q * 2 + g - 6) * 16 + lane
                        if nvalid > 0:
                            pidx[slot][q, pl.ds(g * 16, 16)] = jnp.where(
                                lane < nvalid, v >> 2, dummy)
                        else:
                            pidx[slot][q, pl.ds(g * 16, 16)] = dummy
                    ofb[slot][q, pl.ds(g * 16, 16)] = (v & 3) * D
                    if nvalid >= 16:
                        x = lane + ((r * idx_rows_per_round + q) * FIELDS
                                    + g * 16)
                        bl = ((x >> 2) * 20972) >> 19
                        srow = (osbase + x * PACK
                                + bl * ((FPAD - FIELDS) * PACK))
                    elif nvalid > 0:
                        x = lane + ((r * idx_rows_per_round + q) * FIELDS
                                    + g * 16)
                        bl = ((x >> 2) * 20972) >> 19
                        dmp = dump0 + (q * 2) * (FPAD * PACK) + lane
                        srow = jnp.where(
                            lane < nvalid,
                            osbase + x * PACK
                            + bl * ((FPAD - FIELDS) * PACK),
                            dmp)
                    else:
                        srow = dump0 + (q * 2 + 1) * (FPAD * PACK) + lane
                    sidx[slot][q, pl.ds(g * 16, 16)] = srow

        def fire_gathers(slot):
            for s in range(idx_rows_per_round):
                pltpu.async_copy(
                    table_hbm.at[pidx[slot].at[s]],
                    pk[slot].at[pl.ds(s * LANES, LANES)], sg[slot])

        def wait_gathers(slot):
            pltpu.make_async_copy(
                table_hbm.at[pl.ds(0, ROUND)], pk[slot], sg[slot]).wait()

        def extract(slot):
            def group(g, _):
                off = ofb[slot][g // 8, pl.ds((g % 8) * 16, 16)]
                lj = lane + g * 16
                for c in range(D):
                    rot = (lane + c) & (D - 1)
                    vals = plsc.load_gather(pk[slot], [lj, off + rot])
                    plsc.store_scatter(ob[slot], [lj, rot], vals)
                return 0

            lax.fori_loop(0, GROUPS, group, 0)

        def fire_out(slot):
            for s in range(idx_rows_per_round):
                pltpu.async_copy(
                    ob[slot].at[pl.ds(s * LANES, LANES)],
                    out_hbm.at[sidx[slot].at[s]], so[slot])

        def wait_out(slot):
            pltpu.make_async_copy(
                ob[slot], out_hbm.at[pl.ds(0, ROUND)], so[slot]).wait()

        def step(r, slot, other):
            def advance():
                wait_idx(other)
                pl.when(r >= 1)(lambda: wait_out(other))
                prep(r + 1, other)
            pl.when(r + 1 < n_round)(advance)
            wait_gathers(slot)
            pl.when(r + 1 < n_round)(lambda: fire_gathers(other))
            pl.when(r + 2 < n_round)(lambda: fire_idx(r + 2, slot))
            extract(slot)
            fire_out(slot)

        fire_idx(0, 0)
        wait_idx(0)
        prep(0, 0)
        fire_gathers(0)
        fire_idx(1, 1)

        def pair(i, _):
            r0 = i * 2
            step(r0, 0, 1)
            step(r0 + 1, 1, 0)
            return 0

        lax.fori_loop(0, n_round // 2, pair, 0)
        wait_out(0)
        wait_out(1)

    return body


@jax.jit
def kernel(input, weight):
    B, F = input.shape
    V, _ = weight.shape
    total = B * F
    idx = jnp.pad(input.astype(jnp.int32), ((0, 0), (0, LANES - F)))
    wpk = weight.reshape(V // PACK, LANES)
    out = _make_kernel(total, V)(idx, wpk)
    return out.reshape(B, FPAD, PACK * D)[:, :F, :D]


# speedup vs baseline: 15.0776x; 1.1610x over previous
import functools
import jax
import jax.numpy as jnp
from jax import lax
from jax.experimental import pallas as pl
from jax.experimental.pallas import tpu as pltpu
from jax.experimental.pallas import tpu_sc as plsc

NC = 2
NS = 16
NW = NC * NS

ROUND = 256
PACK = 4
LANES = 128
D = 32
GROUPS = ROUND // 16
FIELDS = 100
FPAD = 104


def _make_kernel(total, vocab):
    per_w = total // NW
    n_round = per_w // ROUND
    idx_rows_per_round = ROUND // LANES
    idx_rows_per_w = per_w // LANES
    batch = total // FIELDS
    out_rows = batch * FPAD * PACK

    mesh = plsc.VectorSubcoreMesh(core_axis_name="c", subcore_axis_name="s")

    @functools.partial(
        pl.kernel,
        out_type=jax.ShapeDtypeStruct((out_rows, D), jnp.float32),
        mesh=mesh,
        scratch_types=[
            pltpu.VMEM((idx_rows_per_round, LANES), jnp.int32),
            pltpu.VMEM((idx_rows_per_round, LANES), jnp.int32),
            pltpu.VMEM((idx_rows_per_round, LANES), jnp.int32),
            pltpu.VMEM((idx_rows_per_round, LANES), jnp.int32),
            pltpu.VMEM((idx_rows_per_round, LANES), jnp.int32),
            pltpu.VMEM((idx_rows_per_round, LANES), jnp.int32),
            pltpu.VMEM((idx_rows_per_round, LANES), jnp.int32),
            pltpu.VMEM((idx_rows_per_round, LANES), jnp.int32),
            pltpu.VMEM((ROUND, LANES), jnp.float32),
            pltpu.VMEM((ROUND, LANES), jnp.float32),
            pltpu.VMEM((ROUND, D), jnp.float32),
            pltpu.VMEM((ROUND, D), jnp.float32),
            pltpu.SemaphoreType.DMA,
            pltpu.SemaphoreType.DMA,
            pltpu.SemaphoreType.DMA,
            pltpu.SemaphoreType.DMA,
            pltpu.SemaphoreType.DMA,
            pltpu.SemaphoreType.DMA,
        ],
        compiler_params=pltpu.CompilerParams(
            use_tc_tiling_on_sc=False, needs_layout_passes=False),
    )
    def body(idx_hbm, table_hbm, out_hbm,
             idxb0, idxb1, pidx0, pidx1, ofb0, ofb1, sidx0, sidx1,
             pk0, pk1, ob0, ob1, si0, si1, sg0, sg1, so0, so1):
        wid = lax.axis_index("s") * NC + lax.axis_index("c")
        ibase = wid * idx_rows_per_w
        osbase = wid * (per_w // FIELDS) * FPAD * PACK
        idxb = (idxb0, idxb1)
        pidx = (pidx0, pidx1)
        ofb = (ofb0, ofb1)
        sidx = (sidx0, sidx1)
        pk = (pk0, pk1)
        ob = (ob0, ob1)
        si = (si0, si1)
        sg = (sg0, sg1)
        so = (so0, so1)
        lane = lax.iota(jnp.int32, 16)

        def fire_idx(r, slot):
            pltpu.async_copy(
                idx_hbm.at[pl.ds(ibase + r * idx_rows_per_round,
                                 idx_rows_per_round)],
                idxb[slot], si[slot])

        def wait_idx(slot):
            pltpu.make_async_copy(
                idx_hbm.at[pl.ds(0, idx_rows_per_round)], idxb[slot],
                si[slot]).wait()

        def prep(r, slot):
            for q in range(idx_rows_per_round):
                for g in range(LANES // 16):
                    v = idxb[slot][q, pl.ds(g * 16, 16)]
                    pidx[slot][q, pl.ds(g * 16, 16)] = v >> 2
                    ofb[slot][q, pl.ds(g * 16, 16)] = (v & 3) * D
                    x = lane + (r * ROUND + q * LANES + g * 16)
                    bl = ((x >> 2) * 20972) >> 19
                    sidx[slot][q, pl.ds(g * 16, 16)] = (
                        osbase + x * PACK + bl * ((FPAD - FIELDS) * PACK))

        def fire_gathers(slot):
            for s in range(idx_rows_per_round):
                pltpu.async_copy(
                    table_hbm.at[pidx[slot].at[s]],
                    pk[slot].at[pl.ds(s * LANES, LANES)], sg[slot])

        def wait_gathers(slot):
            pltpu.make_async_copy(
                table_hbm.at[pl.ds(0, ROUND)], pk[slot], sg[slot]).wait()

        def extract(slot):
            def group(g, _):
                off = ofb[slot][g // 8, pl.ds((g % 8) * 16, 16)]
                lj = lane + g * 16
                for c in range(D):
                    rot = (lane + c) & (D - 1)
                    vals = plsc.load_gather(pk[slot], [lj, off + rot])
                    plsc.store_scatter(ob[slot], [lj, rot], vals)
                return 0

            lax.fori_loop(0, GROUPS, group, 0)

        def fire_out(slot):
            for s in range(idx_rows_per_round):
                pltpu.async_copy(
                    ob[slot].at[pl.ds(s * LANES, LANES)],
                    out_hbm.at[sidx[slot].at[s]], so[slot])

        def wait_out(slot):
            pltpu.make_async_copy(
                ob[slot], out_hbm.at[pl.ds(0, ROUND)], so[slot]).wait()

        def step(r, slot, other):
            def advance():
                wait_idx(other)
                pl.when(r >= 1)(lambda: wait_out(other))
                prep(r + 1, other)
            pl.when(r + 1 < n_round)(advance)
            wait_gathers(slot)
            pl.when(r + 1 < n_round)(lambda: fire_gathers(other))
            pl.when(r + 2 < n_round)(lambda: fire_idx(r + 2, slot))
            extract(slot)
            fire_out(slot)

        fire_idx(0, 0)
        wait_idx(0)
        prep(0, 0)
        fire_gathers(0)
        fire_idx(1, 1)

        def pair(i, _):
            r0 = i * 2
            step(r0, 0, 1)
            step(r0 + 1, 1, 0)
            return 0

        lax.fori_loop(0, n_round // 2, pair, 0)
        wait_out(0)
        wait_out(1)

    return body


@jax.jit
def kernel(input, weight):
    B, F = input.shape
    V, _ = weight.shape
    total = B * F
    idx = input.astype(jnp.int32).reshape(total // LANES, LANES)
    wpk = weight.reshape(V // PACK, LANES)
    out = _make_kernel(total, V)(idx, wpk)
    return out.reshape(B, FPAD, PACK * D)[:, :F, :D]
